# Initial kernel scaffold; baseline (speedup 1.0000x reference)
import jax
import jax.numpy as jnp
from jax.experimental import pallas as pl


def kernel(x, edge_index, W1, W2):
    raise NotImplementedError("write your pallas kernel here")


# speedup vs baseline: 31.5315x; 31.5315x over previous
import functools

import jax
import jax.numpy as jnp
from jax import lax
from jax.experimental import pallas as pl
from jax.experimental.pallas import tpu as pltpu
from jax.experimental.pallas import tpu_sc as plsc

N_NODES = 10000
D_FEAT = 128
CH = 16

NC = 2
NS = 16
NW = NC * NS
B = 128
NPAD = 10240
RPT = NPAD // NS

_MESH = plsc.VectorSubcoreMesh(core_axis_name="c", subcore_axis_name="s")


def _deg_body(dst_hbm, out_hbm, dstbuf, ones_v, zbuf, deg_sh):
    c = lax.axis_index("c")
    s = lax.axis_index("s")
    w = c * NS + s
    T = dstbuf.shape[0]
    pltpu.sync_copy(dst_hbm.at[pl.ds(w * T, T)], dstbuf)

    z16 = jnp.zeros((16,), jnp.float32)
    o16 = jnp.ones((16,), jnp.float32)

    @pl.loop(0, RPT // 16)
    def _zero(i):
        zbuf[pl.ds(i * 16, 16)] = z16

    @pl.loop(0, B // 16)
    def _ones(i):
        ones_v[pl.ds(i * 16, 16)] = o16

    pltpu.sync_copy(zbuf, deg_sh.at[pl.ds(s * RPT, RPT)])
    plsc.subcore_barrier()

    @pl.loop(0, T)
    def _accum(j):
        pltpu.sync_copy(ones_v, deg_sh.at[dstbuf.at[j]], add=True)

    plsc.subcore_barrier()
    pltpu.sync_copy(deg_sh.at[pl.ds(s * RPT, RPT)],
                    out_hbm.at[c, pl.ds(s * RPT, RPT)])


def _agg_body(y_hbm, src_hbm, dst_hbm, out_hbm,
              srcbuf, dstbuf, rows0, acc, sem0):
    c = lax.axis_index("c")
    s = lax.axis_index("s")
    w = c * NS + s
    T = srcbuf.shape[0]
    pltpu.sync_copy(src_hbm.at[pl.ds(w * T, T)], srcbuf)
    pltpu.sync_copy(dst_hbm.at[pl.ds(w * T, T)], dstbuf)

    z16 = jnp.zeros((16,), jnp.float32)

    @pl.loop(0, B)
    def _zero(i):
        rows0[i] = z16

    @pl.loop(0, RPT // B)
    def _init(i):
        pltpu.sync_copy(rows0, acc.at[pl.ds(s * RPT + i * B, B)])

    plsc.subcore_barrier()

    @pl.loop(0, T)
    def _edges(j):
        pltpu.async_copy(y_hbm.at[srcbuf.at[j]], rows0, sem0).wait()
        pltpu.sync_copy(rows0, acc.at[dstbuf.at[j]], add=True)

    plsc.subcore_barrier()
    pltpu.sync_copy(acc.at[pl.ds(s * RPT, RPT)],
                    out_hbm.at[c, pl.ds(s * RPT, RPT)])


def _make_deg_call(T):
    return pl.kernel(
        _deg_body,
        out_type=jax.ShapeDtypeStruct((NC, NPAD), jnp.float32),
        mesh=_MESH,
        scratch_types=[
            pltpu.VMEM((T, B), jnp.int32),
            pltpu.VMEM((B,), jnp.float32),
            pltpu.VMEM((RPT,), jnp.float32),
            pltpu.VMEM_SHARED((NPAD,), jnp.float32),
        ],
    )


def _make_agg_call(T):
    return pl.kernel(
        _agg_body,
        out_type=jax.ShapeDtypeStruct((NC, NPAD, CH), jnp.float32),
        mesh=_MESH,
        scratch_types=[
            pltpu.VMEM((T, B), jnp.int32),
            pltpu.VMEM((T, B), jnp.int32),
            pltpu.VMEM((B, CH), jnp.float32),
            pltpu.VMEM_SHARED((NPAD, CH), jnp.float32),
            pltpu.SemaphoreType.DMA,
        ],
        compiler_params=pltpu.CompilerParams(use_tc_tiling_on_sc=False),
    )


def _tc1_body(x_ref, w1_ref, degp_ref, y1_ref, dinv_ref):
    deg = degp_ref[0] + degp_ref[1] + 1.0
    dinv = 1.0 / jnp.sqrt(deg)
    h = jnp.dot(x_ref[...], w1_ref[...], preferred_element_type=jnp.float32)
    dinv_col = dinv[:, None]
    y1_ref[...] = h * dinv_col
    dinv_ref[...] = dinv_col


def _tc2_body(p_ref, y1_ref, dinv_ref, y2_ref):
    agg = (p_ref[0] + p_ref[1] + y1_ref[...]) * dinv_ref[...]
    y2_ref[...] = jnp.maximum(agg, 0.0) * dinv_ref[...]


def _tc3_body(q_ref, y2_ref, dinv_ref, w2_ref, out_ref):
    z = (q_ref[0] + q_ref[1] + y2_ref[...]) * dinv_ref[...]
    z = jnp.dot(z, w2_ref[...], preferred_element_type=jnp.float32)
    z = z - jnp.max(z, axis=-1, keepdims=True)
    e = jnp.exp(z)
    out_ref[...] = e / jnp.sum(e, axis=-1, keepdims=True)


_tc1 = pl.pallas_call(
    _tc1_body,
    out_shape=[
        jax.ShapeDtypeStruct((NPAD, CH), jnp.float32),
        jax.ShapeDtypeStruct((NPAD, 1), jnp.float32),
    ],
)

_tc2 = pl.pallas_call(
    _tc2_body,
    out_shape=jax.ShapeDtypeStruct((NPAD, CH), jnp.float32),
)

_tc3 = pl.pallas_call(
    _tc3_body,
    out_shape=jax.ShapeDtypeStruct((NPAD, CH), jnp.float32),
)


def kernel(x, edge_index, W1, W2):
    src = edge_index[0].astype(jnp.int32)
    dst = edge_index[1].astype(jnp.int32)
    E = src.shape[0]
    T = -(-E // (NW * B))
    T = T + (T % 2)
    EPAD = NW * B * T

    srcp = jnp.pad(src, (0, EPAD - E), constant_values=N_NODES).reshape(NW * T, B)
    dstp = jnp.pad(dst, (0, EPAD - E), constant_values=N_NODES).reshape(NW * T, B)
    xpad = jnp.pad(x.astype(jnp.float32), ((0, NPAD - N_NODES), (0, 0)))

    deg_call = _make_deg_call(T)
    agg_call = _make_agg_call(T)

    degp = deg_call(dstp)
    y1, dinv = _tc1(xpad, W1, degp)
    p = agg_call(y1, srcp, dstp)
    y2 = _tc2(p, y1, dinv)
    q = agg_call(y2, srcp, dstp)
    out = _tc3(q, y2, dinv, W2)
    return out[:N_NODES]


# speedup vs baseline: 40.5917x; 1.2873x over previous
import functools

import jax
import jax.numpy as jnp
from jax import lax
from jax.experimental import pallas as pl
from jax.experimental.pallas import tpu as pltpu
from jax.experimental.pallas import tpu_sc as plsc

N_NODES = 10000
D_FEAT = 128
CH = 16

NC = 2
NS = 16
NW = NC * NS
B = 128
NPAD = 10240
RPT = NPAD // NS

_MESH = plsc.VectorSubcoreMesh(core_axis_name="c", subcore_axis_name="s")


def _deg_body(dst_hbm, out_hbm, dstbuf, ones_v, zbuf, deg_sh):
    c = lax.axis_index("c")
    s = lax.axis_index("s")
    w = c * NS + s
    T = dstbuf.shape[0]
    pltpu.sync_copy(dst_hbm.at[pl.ds(w * T, T)], dstbuf)

    z16 = jnp.zeros((16,), jnp.float32)
    o16 = jnp.ones((16,), jnp.float32)

    @pl.loop(0, RPT // 16)
    def _zero(i):
        zbuf[pl.ds(i * 16, 16)] = z16

    @pl.loop(0, B // 16)
    def _ones(i):
        ones_v[pl.ds(i * 16, 16)] = o16

    pltpu.sync_copy(zbuf, deg_sh.at[pl.ds(s * RPT, RPT)])
    plsc.subcore_barrier()

    @pl.loop(0, T)
    def _accum(j):
        pltpu.sync_copy(ones_v, deg_sh.at[dstbuf.at[j]], add=True)

    plsc.subcore_barrier()
    pltpu.sync_copy(deg_sh.at[pl.ds(s * RPT, RPT)],
                    out_hbm.at[c, pl.ds(s * RPT, RPT)])


NB = 4


def _agg_body(y_hbm, src_hbm, dst_hbm, out_hbm,
              srcbuf, dstbuf, rows, acc, gsem, ssem):
    c = lax.axis_index("c")
    s = lax.axis_index("s")
    w = c * NS + s
    T = srcbuf.shape[0]
    pltpu.sync_copy(src_hbm.at[pl.ds(w * T, T)], srcbuf)
    pltpu.sync_copy(dst_hbm.at[pl.ds(w * T, T)], dstbuf)

    z16 = jnp.zeros((16,), jnp.float32)

    @pl.loop(0, B)
    def _zero(i):
        rows[0, i] = z16

    @pl.loop(0, RPT // B)
    def _init(i):
        pltpu.sync_copy(rows.at[0], acc.at[pl.ds(s * RPT + i * B, B)])

    plsc.subcore_barrier()

    for b in range(NB):
        pltpu.async_copy(y_hbm.at[srcbuf.at[b]], rows.at[b], gsem.at[b])

    @pl.loop(0, T, step=NB)
    def _edges(jj):
        for b in range(NB):
            j = jj + b
            pltpu.make_async_copy(y_hbm.at[srcbuf.at[j]], rows.at[b],
                                  gsem.at[b]).wait()
            pltpu.async_copy(rows.at[b], acc.at[dstbuf.at[j]], ssem.at[b],
                             add=True)
        for b in range(NB):
            j = jj + b
            pltpu.make_async_copy(rows.at[b], acc.at[dstbuf.at[j]],
                                  ssem.at[b]).wait()

            @pl.when(j + NB < T)
            def _prefetch():
                pltpu.async_copy(y_hbm.at[srcbuf.at[j + NB]], rows.at[b],
                                 gsem.at[b])

    plsc.subcore_barrier()
    pltpu.sync_copy(acc.at[pl.ds(s * RPT, RPT)],
                    out_hbm.at[c, pl.ds(s * RPT, RPT)])


def _make_deg_call(T):
    return pl.kernel(
        _deg_body,
        out_type=jax.ShapeDtypeStruct((NC, NPAD), jnp.float32),
        mesh=_MESH,
        scratch_types=[
            pltpu.VMEM((T, B), jnp.int32),
            pltpu.VMEM((B,), jnp.float32),
            pltpu.VMEM((RPT,), jnp.float32),
            pltpu.VMEM_SHARED((NPAD,), jnp.float32),
        ],
    )


def _make_agg_call(T):
    return pl.kernel(
        _agg_body,
        out_type=jax.ShapeDtypeStruct((NC, NPAD, CH), jnp.float32),
        mesh=_MESH,
        scratch_types=[
            pltpu.VMEM((T, B), jnp.int32),
            pltpu.VMEM((T, B), jnp.int32),
            pltpu.VMEM((NB, B, CH), jnp.float32),
            pltpu.VMEM_SHARED((NPAD, CH), jnp.float32),
            pltpu.SemaphoreType.DMA((NB,)),
            pltpu.SemaphoreType.DMA((NB,)),
        ],
        compiler_params=pltpu.CompilerParams(use_tc_tiling_on_sc=False),
    )


def _tc1_body(x_ref, w1_ref, degp_ref, y1_ref, dinv_ref):
    deg = degp_ref[0] + degp_ref[1] + 1.0
    dinv = 1.0 / jnp.sqrt(deg)
    h = jnp.dot(x_ref[...], w1_ref[...], preferred_element_type=jnp.float32)
    dinv_col = dinv[:, None]
    y1_ref[...] = h * dinv_col
    dinv_ref[...] = dinv_col


def _tc2_body(p_ref, y1_ref, dinv_ref, y2_ref):
    agg = (p_ref[0] + p_ref[1] + y1_ref[...]) * dinv_ref[...]
    y2_ref[...] = jnp.maximum(agg, 0.0) * dinv_ref[...]


def _tc3_body(q_ref, y2_ref, dinv_ref, w2_ref, out_ref):
    z = (q_ref[0] + q_ref[1] + y2_ref[...]) * dinv_ref[...]
    z = jnp.dot(z, w2_ref[...], preferred_element_type=jnp.float32)
    z = z - jnp.max(z, axis=-1, keepdims=True)
    e = jnp.exp(z)
    out_ref[...] = e / jnp.sum(e, axis=-1, keepdims=True)


_tc1 = pl.pallas_call(
    _tc1_body,
    out_shape=[
        jax.ShapeDtypeStruct((NPAD, CH), jnp.float32),
        jax.ShapeDtypeStruct((NPAD, 1), jnp.float32),
    ],
)

_tc2 = pl.pallas_call(
    _tc2_body,
    out_shape=jax.ShapeDtypeStruct((NPAD, CH), jnp.float32),
)

_tc3 = pl.pallas_call(
    _tc3_body,
    out_shape=jax.ShapeDtypeStruct((NPAD, CH), jnp.float32),
)


def kernel(x, edge_index, W1, W2):
    src = edge_index[0].astype(jnp.int32)
    dst = edge_index[1].astype(jnp.int32)
    E = src.shape[0]
    T = -(-E // (NW * B))
    T = -(-T // NB) * NB
    EPAD = NW * B * T

    pad_dst = N_NODES + (jnp.arange(EPAD - E, dtype=jnp.int32) % (NPAD - N_NODES))
    srcp = jnp.pad(src, (0, EPAD - E), constant_values=N_NODES).reshape(NW * T, B)
    dstp = jnp.concatenate([dst, pad_dst]).reshape(NW * T, B)
    xpad = jnp.pad(x.astype(jnp.float32), ((0, NPAD - N_NODES), (0, 0)))

    deg_call = _make_deg_call(T)
    agg_call = _make_agg_call(T)

    degp = deg_call(dstp)
    y1, dinv = _tc1(xpad, W1, degp)
    p = agg_call(y1, srcp, dstp)
    y2 = _tc2(p, y1, dinv)
    q = agg_call(y2, srcp, dstp)
    out = _tc3(q, y2, dinv, W2)
    return out[:N_NODES]


# speedup vs baseline: 62.3772x; 1.5367x over previous
import functools

import jax
import jax.numpy as jnp
from jax import lax
from jax.experimental import pallas as pl
from jax.experimental.pallas import tpu as pltpu
from jax.experimental.pallas import tpu_sc as plsc

N_NODES = 10000
D_FEAT = 128
CH = 16

NC = 2
NS = 16
NW = NC * NS
B = 128
NPAD = 10240
RPT = NPAD // NS

_MESH = plsc.VectorSubcoreMesh(core_axis_name="c", subcore_axis_name="s")


def _deg_body(dst_hbm, out_hbm, dstbuf, ones_v, zbuf, deg_sh, dsem):
    c = lax.axis_index("c")
    s = lax.axis_index("s")
    w = c * NS + s
    T = dstbuf.shape[0]
    pltpu.sync_copy(dst_hbm.at[pl.ds(w * T, T)], dstbuf)

    z16 = jnp.zeros((16,), jnp.float32)
    o16 = jnp.ones((16,), jnp.float32)

    @pl.loop(0, RPT // 16)
    def _zero(i):
        zbuf[pl.ds(i * 16, 16)] = z16

    @pl.loop(0, B // 16)
    def _ones(i):
        ones_v[pl.ds(i * 16, 16)] = o16

    pltpu.sync_copy(zbuf, deg_sh.at[pl.ds(s * RPT, RPT)])
    plsc.subcore_barrier()

    K = 16

    @pl.loop(0, T, step=K)
    def _accum(jj):
        for b in range(K):
            pltpu.async_copy(ones_v, deg_sh.at[dstbuf.at[jj + b]], dsem,
                             add=True)
        for b in range(K):
            pltpu.make_async_copy(ones_v, deg_sh.at[dstbuf.at[jj + b]],
                                  dsem).wait()

    plsc.subcore_barrier()
    pltpu.sync_copy(deg_sh.at[pl.ds(s * RPT, RPT)],
                    out_hbm.at[c, pl.ds(s * RPT, RPT)])


NB = 4


def _agg_body(y_hbm, src_hbm, dst_hbm, out_hbm,
              srcbuf, dstbuf, rows, acc, gsem, ssem):
    c = lax.axis_index("c")
    s = lax.axis_index("s")
    w = c * NS + s
    T = srcbuf.shape[0]
    pltpu.sync_copy(src_hbm.at[pl.ds(w * T, T)], srcbuf)
    pltpu.sync_copy(dst_hbm.at[pl.ds(w * T, T)], dstbuf)

    z16 = jnp.zeros((16,), jnp.float32)

    @pl.loop(0, B)
    def _zero(i):
        rows[0, i] = z16

    @pl.loop(0, RPT // B)
    def _init(i):
        pltpu.sync_copy(rows.at[0], acc.at[pl.ds(s * RPT + i * B, B)])

    plsc.subcore_barrier()

    for b in range(NB):
        pltpu.async_copy(y_hbm.at[srcbuf.at[b]], rows.at[b], gsem.at[b])

    @pl.loop(0, T, step=NB)
    def _edges(jj):
        for b in range(NB):
            j = jj + b
            pltpu.make_async_copy(y_hbm.at[srcbuf.at[j]], rows.at[b],
                                  gsem.at[b]).wait()
            pltpu.async_copy(rows.at[b], acc.at[dstbuf.at[j]], ssem.at[b],
                             add=True)
        for b in range(NB):
            j = jj + b
            pltpu.make_async_copy(rows.at[b], acc.at[dstbuf.at[j]],
                                  ssem.at[b]).wait()

            @pl.when(j + NB < T)
            def _prefetch():
                pltpu.async_copy(y_hbm.at[srcbuf.at[j + NB]], rows.at[b],
                                 gsem.at[b])

    plsc.subcore_barrier()
    pltpu.sync_copy(acc.at[pl.ds(s * RPT, RPT)],
                    out_hbm.at[c, pl.ds(s * RPT, RPT)])


def _make_deg_call(T):
    return pl.kernel(
        _deg_body,
        out_type=jax.ShapeDtypeStruct((NC, NPAD), jnp.float32),
        mesh=_MESH,
        scratch_types=[
            pltpu.VMEM((T, B), jnp.int32),
            pltpu.VMEM((B,), jnp.float32),
            pltpu.VMEM((RPT,), jnp.float32),
            pltpu.VMEM_SHARED((NPAD,), jnp.float32),
            pltpu.SemaphoreType.DMA,
        ],
    )


def _make_agg_call(T):
    return pl.kernel(
        _agg_body,
        out_type=jax.ShapeDtypeStruct((NC, NPAD, CH), jnp.float32),
        mesh=_MESH,
        scratch_types=[
            pltpu.VMEM((T, B), jnp.int32),
            pltpu.VMEM((T, B), jnp.int32),
            pltpu.VMEM((NB, B, CH), jnp.float32),
            pltpu.VMEM_SHARED((NPAD, CH), jnp.float32),
            pltpu.SemaphoreType.DMA((NB,)),
            pltpu.SemaphoreType.DMA((NB,)),
        ],
        compiler_params=pltpu.CompilerParams(use_tc_tiling_on_sc=False),
    )


def _tc1_body(x_ref, w1_ref, degp_ref, y1_ref, dinv_ref):
    deg = degp_ref[0] + degp_ref[1] + 1.0
    dinv = 1.0 / jnp.sqrt(deg)
    h = jnp.dot(x_ref[...], w1_ref[...], preferred_element_type=jnp.float32)
    dinv_col = dinv[:, None]
    y1_ref[...] = h * dinv_col
    dinv_ref[...] = dinv_col


def _tc2_body(p_ref, y1_ref, dinv_ref, y2_ref):
    agg = (p_ref[0] + p_ref[1] + y1_ref[...]) * dinv_ref[...]
    y2_ref[...] = jnp.maximum(agg, 0.0) * dinv_ref[...]


def _tc3_body(q_ref, y2_ref, dinv_ref, w2_ref, out_ref):
    z = (q_ref[0] + q_ref[1] + y2_ref[...]) * dinv_ref[...]
    z = jnp.dot(z, w2_ref[...], preferred_element_type=jnp.float32)
    z = z - jnp.max(z, axis=-1, keepdims=True)
    e = jnp.exp(z)
    out_ref[...] = e / jnp.sum(e, axis=-1, keepdims=True)


_tc1 = pl.pallas_call(
    _tc1_body,
    out_shape=[
        jax.ShapeDtypeStruct((NPAD, CH), jnp.float32),
        jax.ShapeDtypeStruct((NPAD, 1), jnp.float32),
    ],
)

_tc2 = pl.pallas_call(
    _tc2_body,
    out_shape=jax.ShapeDtypeStruct((NPAD, CH), jnp.float32),
)

_tc3 = pl.pallas_call(
    _tc3_body,
    out_shape=jax.ShapeDtypeStruct((NPAD, CH), jnp.float32),
)


def kernel(x, edge_index, W1, W2):
    src = edge_index[0].astype(jnp.int32)
    dst = edge_index[1].astype(jnp.int32)
    E = src.shape[0]
    T = -(-E // (NW * B))
    T = -(-T // NB) * NB
    EPAD = NW * B * T

    pad_idx = N_NODES + (jnp.arange(EPAD - E, dtype=jnp.int32) % (NPAD - N_NODES))
    srcp = jnp.concatenate([src, pad_idx]).reshape(NW * T, B)
    dstp = jnp.concatenate([dst, pad_idx]).reshape(NW * T, B)
    xpad = jnp.pad(x.astype(jnp.float32), ((0, NPAD - N_NODES), (0, 0)))

    deg_call = _make_deg_call(T)
    agg_call = _make_agg_call(T)

    degp = deg_call(dstp)
    y1, dinv = _tc1(xpad, W1, degp)
    p = agg_call(y1, srcp, dstp)
    y2 = _tc2(p, y1, dinv)
    q = agg_call(y2, srcp, dstp)
    out = _tc3(q, y2, dinv, W2)
    return out[:N_NODES]


# speedup vs baseline: 64.4724x; 1.0336x over previous
import functools

import jax
import jax.numpy as jnp
from jax import lax
from jax.experimental import pallas as pl
from jax.experimental.pallas import tpu as pltpu
from jax.experimental.pallas import tpu_sc as plsc

N_NODES = 10000
D_FEAT = 128
CH = 16

NC = 2
NS = 16
NW = NC * NS
B = 128
NPAD = 10240
RPT = NPAD // NS

_MESH = plsc.VectorSubcoreMesh(core_axis_name="c", subcore_axis_name="s")


def _deg_body(dst_hbm, out_hbm, dstbuf, ones_v, zbuf, deg_sh, dsem):
    c = lax.axis_index("c")
    s = lax.axis_index("s")
    w = c * NS + s
    T = dstbuf.shape[0]
    pltpu.sync_copy(dst_hbm.at[pl.ds(w * T, T)], dstbuf)

    z16 = jnp.zeros((16,), jnp.float32)
    o16 = jnp.ones((16,), jnp.float32)

    @pl.loop(0, RPT // 16)
    def _zero(i):
        zbuf[pl.ds(i * 16, 16)] = z16

    @pl.loop(0, B // 16)
    def _ones(i):
        ones_v[pl.ds(i * 16, 16)] = o16

    pltpu.sync_copy(zbuf, deg_sh.at[pl.ds(s * RPT, RPT)])
    plsc.subcore_barrier()

    K = 16

    @pl.loop(0, T, step=K)
    def _accum(jj):
        for b in range(K):
            pltpu.async_copy(ones_v, deg_sh.at[dstbuf.at[jj + b]], dsem,
                             add=True)
        for b in range(K):
            pltpu.make_async_copy(ones_v, deg_sh.at[dstbuf.at[jj + b]],
                                  dsem).wait()

    plsc.subcore_barrier()
    pltpu.sync_copy(deg_sh.at[pl.ds(s * RPT, RPT)],
                    out_hbm.at[c, pl.ds(s * RPT, RPT)])


NB = 8


def _agg_body(y_hbm, src_hbm, dst_hbm, out_hbm,
              srcbuf, dstbuf, rows, acc, gsem, ssem):
    c = lax.axis_index("c")
    s = lax.axis_index("s")
    w = c * NS + s
    T = srcbuf.shape[0]
    pltpu.sync_copy(src_hbm.at[pl.ds(w * T, T)], srcbuf)
    pltpu.sync_copy(dst_hbm.at[pl.ds(w * T, T)], dstbuf)

    z16 = jnp.zeros((16,), jnp.float32)

    @pl.loop(0, B)
    def _zero(i):
        rows[0, i] = z16

    @pl.loop(0, RPT // B)
    def _init(i):
        pltpu.sync_copy(rows.at[0], acc.at[pl.ds(s * RPT + i * B, B)])

    plsc.subcore_barrier()

    for b in range(NB):
        pltpu.async_copy(y_hbm.at[srcbuf.at[b]], rows.at[b], gsem.at[b])

    @pl.loop(0, T, step=NB)
    def _edges(jj):
        for b in range(NB):
            j = jj + b
            pltpu.make_async_copy(y_hbm.at[srcbuf.at[j]], rows.at[b],
                                  gsem.at[b]).wait()
            pltpu.async_copy(rows.at[b], acc.at[dstbuf.at[j]], ssem.at[b],
                             add=True)
        for b in range(NB):
            j = jj + b
            pltpu.make_async_copy(rows.at[b], acc.at[dstbuf.at[j]],
                                  ssem.at[b]).wait()

            @pl.when(j + NB < T)
            def _prefetch():
                pltpu.async_copy(y_hbm.at[srcbuf.at[j + NB]], rows.at[b],
                                 gsem.at[b])

    plsc.subcore_barrier()
    pltpu.sync_copy(acc.at[pl.ds(s * RPT, RPT)],
                    out_hbm.at[c, pl.ds(s * RPT, RPT)])


def _make_deg_call(T):
    return pl.kernel(
        _deg_body,
        out_type=jax.ShapeDtypeStruct((NC, NPAD), jnp.float32),
        mesh=_MESH,
        scratch_types=[
            pltpu.VMEM((T, B), jnp.int32),
            pltpu.VMEM((B,), jnp.float32),
            pltpu.VMEM((RPT,), jnp.float32),
            pltpu.VMEM_SHARED((NPAD,), jnp.float32),
            pltpu.SemaphoreType.DMA,
        ],
    )


def _make_agg_call(T):
    return pl.kernel(
        _agg_body,
        out_type=jax.ShapeDtypeStruct((NC, NPAD, CH), jnp.float32),
        mesh=_MESH,
        scratch_types=[
            pltpu.VMEM((T, B), jnp.int32),
            pltpu.VMEM((T, B), jnp.int32),
            pltpu.VMEM((NB, B, CH), jnp.float32),
            pltpu.VMEM_SHARED((NPAD, CH), jnp.float32),
            pltpu.SemaphoreType.DMA((NB,)),
            pltpu.SemaphoreType.DMA((NB,)),
        ],
        compiler_params=pltpu.CompilerParams(use_tc_tiling_on_sc=False),
    )


_GRID = 8
_RB = NPAD // _GRID


def _tc0_body(x_ref, w1_ref, h_ref):
    h_ref[...] = jnp.dot(x_ref[...], w1_ref[...],
                         preferred_element_type=jnp.float32)


def _tc1_body(h_ref, degp_ref, y1_ref, dinv_ref):
    deg = degp_ref[0] + degp_ref[1] + 1.0
    dinv = 1.0 / jnp.sqrt(deg)
    dinv_col = dinv[:, None]
    y1_ref[...] = h_ref[...] * dinv_col
    dinv_ref[...] = dinv_col


def _tc2_body(p_ref, y1_ref, dinv_ref, y2_ref):
    agg = (p_ref[0] + p_ref[1] + y1_ref[...]) * dinv_ref[...]
    y2_ref[...] = jnp.maximum(agg, 0.0) * dinv_ref[...]


def _tc3_body(q_ref, y2_ref, dinv_ref, w2_ref, out_ref):
    z = (q_ref[0] + q_ref[1] + y2_ref[...]) * dinv_ref[...]
    z = jnp.dot(z, w2_ref[...], preferred_element_type=jnp.float32)
    z = z - jnp.max(z, axis=-1, keepdims=True)
    e = jnp.exp(z)
    out_ref[...] = e / jnp.sum(e, axis=-1, keepdims=True)


def _row_spec(cols):
    return pl.BlockSpec((_RB, cols), lambda i: (i, 0))


def _part_spec(cols):
    return pl.BlockSpec((NC, _RB, cols), lambda i: (0, i, 0))


_tc0 = pl.pallas_call(
    _tc0_body,
    grid=(_GRID,),
    in_specs=[_row_spec(D_FEAT), pl.BlockSpec((D_FEAT, CH), lambda i: (0, 0))],
    out_specs=_row_spec(CH),
    out_shape=jax.ShapeDtypeStruct((NPAD, CH), jnp.float32),
)

_tc1 = pl.pallas_call(
    _tc1_body,
    grid=(_GRID,),
    in_specs=[_row_spec(CH), pl.BlockSpec((NC, _RB), lambda i: (0, i))],
    out_specs=[_row_spec(CH), _row_spec(1)],
    out_shape=[
        jax.ShapeDtypeStruct((NPAD, CH), jnp.float32),
        jax.ShapeDtypeStruct((NPAD, 1), jnp.float32),
    ],
)

_tc2 = pl.pallas_call(
    _tc2_body,
    grid=(_GRID,),
    in_specs=[_part_spec(CH), _row_spec(CH), _row_spec(1)],
    out_specs=_row_spec(CH),
    out_shape=jax.ShapeDtypeStruct((NPAD, CH), jnp.float32),
)

_tc3 = pl.pallas_call(
    _tc3_body,
    grid=(_GRID,),
    in_specs=[_part_spec(CH), _row_spec(CH), _row_spec(1),
              pl.BlockSpec((CH, CH), lambda i: (0, 0))],
    out_specs=_row_spec(CH),
    out_shape=jax.ShapeDtypeStruct((NPAD, CH), jnp.float32),
)


def kernel(x, edge_index, W1, W2):
    src = edge_index[0].astype(jnp.int32)
    dst = edge_index[1].astype(jnp.int32)
    E = src.shape[0]
    T = -(-E // (NW * B))
    T = -(-T // NB) * NB
    EPAD = NW * B * T

    pad_idx = N_NODES + (jnp.arange(EPAD - E, dtype=jnp.int32) % (NPAD - N_NODES))
    srcp = jnp.concatenate([src, pad_idx]).reshape(NW * T, B)
    dstp = jnp.concatenate([dst, pad_idx]).reshape(NW * T, B)
    xpad = jnp.pad(x.astype(jnp.float32), ((0, NPAD - N_NODES), (0, 0)))

    deg_call = _make_deg_call(T)
    agg_call = _make_agg_call(T)

    h = _tc0(xpad, W1)
    degp = deg_call(dstp)
    y1, dinv = _tc1(h, degp)
    p = agg_call(y1, srcp, dstp)
    y2 = _tc2(p, y1, dinv)
    q = agg_call(y2, srcp, dstp)
    out = _tc3(q, y2, dinv, W2)
    return out[:N_NODES]


# speedup vs baseline: 65.0426x; 1.0088x over previous
import functools

import jax
import jax.numpy as jnp
from jax import lax
from jax.experimental import pallas as pl
from jax.experimental.pallas import tpu as pltpu
from jax.experimental.pallas import tpu_sc as plsc

N_NODES = 10000
D_FEAT = 128
CH = 16

NC = 2
NS = 16
NW = NC * NS
B = 128
NPAD = 10240
RPT = NPAD // NS

_MESH = plsc.VectorSubcoreMesh(core_axis_name="c", subcore_axis_name="s")


def _deg_body(dst_hbm, out_hbm, dstbuf, ones_v, zbuf, deg_sh, dsem):
    c = lax.axis_index("c")
    s = lax.axis_index("s")
    w = c * NS + s
    T = dstbuf.shape[0]
    pltpu.sync_copy(dst_hbm.at[pl.ds(w * T, T)], dstbuf)

    z16 = jnp.zeros((16,), jnp.float32)
    o16 = jnp.ones((16,), jnp.float32)

    @pl.loop(0, RPT // 16)
    def _zero(i):
        zbuf[pl.ds(i * 16, 16)] = z16

    @pl.loop(0, B // 16)
    def _ones(i):
        ones_v[pl.ds(i * 16, 16)] = o16

    pltpu.sync_copy(zbuf, deg_sh.at[pl.ds(s * RPT, RPT)])
    plsc.subcore_barrier()

    K = 16

    @pl.loop(0, T, step=K)
    def _accum(jj):
        for b in range(K):
            pltpu.async_copy(ones_v, deg_sh.at[dstbuf.at[jj + b]], dsem,
                             add=True)
        for b in range(K):
            pltpu.make_async_copy(ones_v, deg_sh.at[dstbuf.at[jj + b]],
                                  dsem).wait()

    plsc.subcore_barrier()
    pltpu.sync_copy(deg_sh.at[pl.ds(s * RPT, RPT)],
                    out_hbm.at[c, pl.ds(s * RPT, RPT)])


NB = 8


def _agg_body(y_hbm, src_hbm, dst_hbm, out_hbm,
              srcbuf, dstbuf, rows, acc, gsem, ssem):
    c = lax.axis_index("c")
    s = lax.axis_index("s")
    w = c * NS + s
    T = srcbuf.shape[0]
    pltpu.sync_copy(src_hbm.at[pl.ds(w * T, T)], srcbuf)
    pltpu.sync_copy(dst_hbm.at[pl.ds(w * T, T)], dstbuf)

    z16 = jnp.zeros((16,), jnp.float32)

    @pl.loop(0, B)
    def _zero(i):
        rows[0, i] = z16

    @pl.loop(0, RPT // B)
    def _init(i):
        pltpu.sync_copy(rows.at[0], acc.at[pl.ds(s * RPT + i * B, B)])

    plsc.subcore_barrier()

    for b in range(NB):
        pltpu.async_copy(y_hbm.at[srcbuf.at[b]], rows.at[b], gsem.at[b])

    @pl.loop(0, T, step=NB)
    def _edges(jj):
        for b in range(NB):
            j = jj + b
            pltpu.make_async_copy(y_hbm.at[srcbuf.at[j]], rows.at[b],
                                  gsem.at[b]).wait()
            pltpu.async_copy(rows.at[b], acc.at[dstbuf.at[j]], ssem.at[b],
                             add=True)
        for b in range(NB):
            j = jj + b
            pltpu.make_async_copy(rows.at[b], acc.at[dstbuf.at[j]],
                                  ssem.at[b]).wait()

            @pl.when(j + NB < T)
            def _prefetch():
                pltpu.async_copy(y_hbm.at[srcbuf.at[j + NB]], rows.at[b],
                                 gsem.at[b])

    plsc.subcore_barrier()
    pltpu.sync_copy(acc.at[pl.ds(s * RPT, RPT)],
                    out_hbm.at[c, pl.ds(s * RPT, RPT)])


def _make_deg_call(T):
    return pl.kernel(
        _deg_body,
        out_type=jax.ShapeDtypeStruct((NC, NPAD), jnp.float32),
        mesh=_MESH,
        scratch_types=[
            pltpu.VMEM((T, B), jnp.int32),
            pltpu.VMEM((B,), jnp.float32),
            pltpu.VMEM((RPT,), jnp.float32),
            pltpu.VMEM_SHARED((NPAD,), jnp.float32),
            pltpu.SemaphoreType.DMA,
        ],
    )


def _make_agg_call(T):
    return pl.kernel(
        _agg_body,
        out_type=jax.ShapeDtypeStruct((NC, NPAD, CH), jnp.float32),
        mesh=_MESH,
        scratch_types=[
            pltpu.VMEM((T, B), jnp.int32),
            pltpu.VMEM((T, B), jnp.int32),
            pltpu.VMEM((NB, B, CH), jnp.float32),
            pltpu.VMEM_SHARED((NPAD, CH), jnp.float32),
            pltpu.SemaphoreType.DMA((NB,)),
            pltpu.SemaphoreType.DMA((NB,)),
        ],
        compiler_params=pltpu.CompilerParams(use_tc_tiling_on_sc=False),
    )


_GRID = 8
_RB = NPAD // _GRID


def _tc0_body(x_ref, w1_ref, h_ref):
    h_ref[...] = jnp.dot(x_ref[...], w1_ref[...],
                         preferred_element_type=jnp.float32)


def _tc1_body(h_ref, degp_ref, y1_ref, dinv_ref):
    deg = degp_ref[0] + degp_ref[1] + 1.0
    dinv = 1.0 / jnp.sqrt(deg)
    dinv_col = dinv[:, None]
    y1_ref[...] = h_ref[...] * dinv_col
    dinv_ref[...] = dinv_col


def _tc2_body(p_ref, y1_ref, dinv_ref, y2_ref):
    agg = (p_ref[0] + p_ref[1] + y1_ref[...]) * dinv_ref[...]
    y2_ref[...] = jnp.maximum(agg, 0.0) * dinv_ref[...]


def _tc3_body(q_ref, y2_ref, dinv_ref, w2_ref, out_ref):
    z = (q_ref[0] + q_ref[1] + y2_ref[...]) * dinv_ref[...]
    z = jnp.dot(z, w2_ref[...], preferred_element_type=jnp.float32)
    z = z - jnp.max(z, axis=-1, keepdims=True)
    e = jnp.exp(z)
    out_ref[...] = e / jnp.sum(e, axis=-1, keepdims=True)


_tc0 = pl.pallas_call(
    _tc0_body,
    grid=(_GRID,),
    in_specs=[pl.BlockSpec((_RB, D_FEAT), lambda i: (i, 0)),
              pl.BlockSpec((D_FEAT, CH), lambda i: (0, 0))],
    out_specs=pl.BlockSpec((_RB, CH), lambda i: (i, 0)),
    out_shape=jax.ShapeDtypeStruct((NPAD, CH), jnp.float32),
)

_tc1 = pl.pallas_call(
    _tc1_body,
    out_shape=[
        jax.ShapeDtypeStruct((NPAD, CH), jnp.float32),
        jax.ShapeDtypeStruct((NPAD, 1), jnp.float32),
    ],
)

_tc2 = pl.pallas_call(
    _tc2_body,
    out_shape=jax.ShapeDtypeStruct((NPAD, CH), jnp.float32),
)

_tc3 = pl.pallas_call(
    _tc3_body,
    out_shape=jax.ShapeDtypeStruct((NPAD, CH), jnp.float32),
)


def kernel(x, edge_index, W1, W2):
    src = edge_index[0].astype(jnp.int32)
    dst = edge_index[1].astype(jnp.int32)
    E = src.shape[0]
    T = -(-E // (NW * B))
    T = -(-T // NB) * NB
    EPAD = NW * B * T

    pad_idx = N_NODES + (jnp.arange(EPAD - E, dtype=jnp.int32) % (NPAD - N_NODES))
    srcp = jnp.concatenate([src, pad_idx]).reshape(NW * T, B)
    dstp = jnp.concatenate([dst, pad_idx]).reshape(NW * T, B)
    xpad = jnp.pad(x.astype(jnp.float32), ((0, NPAD - N_NODES), (0, 0)))

    deg_call = _make_deg_call(T)
    agg_call = _make_agg_call(T)

    h = _tc0(xpad, W1)
    degp = deg_call(dstp)
    y1, dinv = _tc1(h, degp)
    p = agg_call(y1, srcp, dstp)
    y2 = _tc2(p, y1, dinv)
    q = agg_call(y2, srcp, dstp)
    out = _tc3(q, y2, dinv, W2)
    return out[:N_NODES]


# speedup vs baseline: 88.3421x; 1.3582x over previous
import functools

import jax
import jax.numpy as jnp
from jax import lax
from jax.experimental import pallas as pl
from jax.experimental.pallas import tpu as pltpu
from jax.experimental.pallas import tpu_sc as plsc

N_NODES = 10000
D_FEAT = 128
CH = 16

NC = 2
NS = 16
NW = NC * NS
B = 128
NPAD = 10240
RPT = NPAD // NS

_MESH = plsc.VectorSubcoreMesh(core_axis_name="c", subcore_axis_name="s")


def _deg_body(dst_hbm, out_hbm, dstbuf, ones_v, zbuf, degbuf, bbuf, deg_sh,
              dsem):
    c = lax.axis_index("c")
    s = lax.axis_index("s")
    w = c * NS + s
    T = dstbuf.shape[0]
    pltpu.sync_copy(dst_hbm.at[pl.ds(w * T, T)], dstbuf)

    z16 = jnp.zeros((16,), jnp.float32)
    o16 = jnp.ones((16,), jnp.float32)

    @pl.loop(0, RPT // 16)
    def _zero(i):
        zbuf[pl.ds(i * 16, 16)] = z16

    @pl.loop(0, B // 16)
    def _ones(i):
        ones_v[pl.ds(i * 16, 16)] = o16

    pltpu.sync_copy(zbuf, deg_sh.at[pl.ds(s * RPT, RPT)])
    plsc.subcore_barrier()

    K = 16

    @pl.loop(0, T, step=K)
    def _accum(jj):
        for b in range(K):
            pltpu.async_copy(ones_v, deg_sh.at[dstbuf.at[jj + b]], dsem,
                             add=True)
        for b in range(K):
            pltpu.make_async_copy(ones_v, deg_sh.at[dstbuf.at[jj + b]],
                                  dsem).wait()

    plsc.subcore_barrier()
    pltpu.sync_copy(deg_sh.at[pl.ds(s * RPT, RPT)], degbuf)

    @pl.loop(0, RPT)
    def _bcast(v):
        val16 = plsc.load_gather(degbuf, [jnp.full((16,), v, jnp.int32)])
        bbuf[v // 8, pl.ds((v % 8) * 16, 16)] = val16

    FR = RPT * CH // 128
    pltpu.sync_copy(bbuf, out_hbm.at[c, pl.ds(s * FR, FR)])


NB = 8


def _agg_body(y_hbm, src_hbm, dst_hbm, out_hbm,
              srcbuf, dstbuf, rows, acc, gsem, ssem):
    c = lax.axis_index("c")
    s = lax.axis_index("s")
    w = c * NS + s
    T = srcbuf.shape[0]
    pltpu.sync_copy(src_hbm.at[pl.ds(w * T, T)], srcbuf)
    pltpu.sync_copy(dst_hbm.at[pl.ds(w * T, T)], dstbuf)

    z16 = jnp.zeros((16,), jnp.float32)

    @pl.loop(0, B)
    def _zero(i):
        rows[0, i] = z16

    @pl.loop(0, RPT // B)
    def _init(i):
        pltpu.sync_copy(rows.at[0], acc.at[pl.ds(s * RPT + i * B, B)])

    plsc.subcore_barrier()

    for b in range(NB):
        pltpu.async_copy(y_hbm.at[srcbuf.at[b]], rows.at[b], gsem.at[b])

    @pl.loop(0, T, step=NB)
    def _edges(jj):
        for b in range(NB):
            j = jj + b
            pltpu.make_async_copy(y_hbm.at[srcbuf.at[j]], rows.at[b],
                                  gsem.at[b]).wait()
            pltpu.async_copy(rows.at[b], acc.at[dstbuf.at[j]], ssem.at[b],
                             add=True)
        for b in range(NB):
            j = jj + b
            pltpu.make_async_copy(rows.at[b], acc.at[dstbuf.at[j]],
                                  ssem.at[b]).wait()

            @pl.when(j + NB < T)
            def _prefetch():
                pltpu.async_copy(y_hbm.at[srcbuf.at[j + NB]], rows.at[b],
                                 gsem.at[b])

    plsc.subcore_barrier()
    pltpu.sync_copy(acc.at[pl.ds(s * RPT, RPT)],
                    out_hbm.at[c, pl.ds(s * RPT, RPT)])


def _make_deg_call(T):
    return pl.kernel(
        _deg_body,
        out_type=jax.ShapeDtypeStruct((NC, NPAD * CH // 128, 128),
                                      jnp.float32),
        mesh=_MESH,
        scratch_types=[
            pltpu.VMEM((T, B), jnp.int32),
            pltpu.VMEM((B,), jnp.float32),
            pltpu.VMEM((RPT,), jnp.float32),
            pltpu.VMEM((RPT,), jnp.float32),
            pltpu.VMEM((RPT * CH // 128, 128), jnp.float32),
            pltpu.VMEM_SHARED((NPAD,), jnp.float32),
            pltpu.SemaphoreType.DMA,
        ],
        compiler_params=pltpu.CompilerParams(needs_layout_passes=False),
    )


def _make_agg_call(T):
    return pl.kernel(
        _agg_body,
        out_type=jax.ShapeDtypeStruct((NC, NPAD, CH), jnp.float32),
        mesh=_MESH,
        scratch_types=[
            pltpu.VMEM((T, B), jnp.int32),
            pltpu.VMEM((T, B), jnp.int32),
            pltpu.VMEM((NB, B, CH), jnp.float32),
            pltpu.VMEM_SHARED((NPAD, CH), jnp.float32),
            pltpu.SemaphoreType.DMA((NB,)),
            pltpu.SemaphoreType.DMA((NB,)),
        ],
        compiler_params=pltpu.CompilerParams(use_tc_tiling_on_sc=False),
    )


_GRID = 8
_RB = NPAD // _GRID
FLAT = (NPAD * CH // 128, 128)


def _tc0_body(x_ref, w1_ref, h_ref):
    h_ref[...] = jnp.dot(x_ref[...], w1_ref[...],
                         preferred_element_type=jnp.float32)


def _tc1_body(h_ref, degb_ref, y1_ref, dinvb_ref):
    dinvb = 1.0 / jnp.sqrt(degb_ref[0] + degb_ref[1] + 1.0)
    y1_ref[...] = h_ref[...] * dinvb
    dinvb_ref[...] = dinvb


def _tc2_body(p_ref, y1_ref, dinvb_ref, y2_ref):
    agg = (p_ref[0] + p_ref[1] + y1_ref[...]) * dinvb_ref[...]
    y2_ref[...] = jnp.maximum(agg, 0.0) * dinvb_ref[...]


def _tc3_body(q_ref, y2_ref, dinvb_ref, w2bd_ref, gsum_ref, out_ref):
    zf = (q_ref[0] + q_ref[1] + y2_ref[...]) * dinvb_ref[...]
    z = jnp.dot(zf, w2bd_ref[...], preferred_element_type=jnp.float32)
    e = jnp.exp(z - jnp.max(z, axis=-1, keepdims=True))
    denom = jnp.dot(e, gsum_ref[...], preferred_element_type=jnp.float32)
    out_ref[...] = e / denom


_tc0 = pl.pallas_call(
    _tc0_body,
    grid=(_GRID,),
    in_specs=[pl.BlockSpec((_RB, D_FEAT), lambda i: (i, 0)),
              pl.BlockSpec((D_FEAT, CH), lambda i: (0, 0))],
    out_specs=pl.BlockSpec((_RB, CH), lambda i: (i, 0)),
    out_shape=jax.ShapeDtypeStruct((NPAD, CH), jnp.float32),
)

_tc1 = pl.pallas_call(
    _tc1_body,
    out_shape=[
        jax.ShapeDtypeStruct(FLAT, jnp.float32),
        jax.ShapeDtypeStruct(FLAT, jnp.float32),
    ],
)

_tc2 = pl.pallas_call(
    _tc2_body,
    out_shape=jax.ShapeDtypeStruct(FLAT, jnp.float32),
)

_tc3 = pl.pallas_call(
    _tc3_body,
    out_shape=jax.ShapeDtypeStruct(FLAT, jnp.float32),
)


def kernel(x, edge_index, W1, W2):
    src = edge_index[0].astype(jnp.int32)
    dst = edge_index[1].astype(jnp.int32)
    E = src.shape[0]
    T = -(-E // (NW * B))
    T = -(-T // NB) * NB
    EPAD = NW * B * T

    pad_idx = N_NODES + (jnp.arange(EPAD - E, dtype=jnp.int32) % (NPAD - N_NODES))
    srcp = jnp.concatenate([src, pad_idx]).reshape(NW * T, B)
    dstp = jnp.concatenate([dst, pad_idx]).reshape(NW * T, B)
    xpad = jnp.pad(x.astype(jnp.float32), ((0, NPAD - N_NODES), (0, 0)))

    deg_call = _make_deg_call(T)
    agg_call = _make_agg_call(T)

    w2bd = jnp.kron(jnp.eye(8, dtype=jnp.float32), W2)
    gsum = jnp.kron(jnp.eye(8, dtype=jnp.float32),
                    jnp.ones((CH, CH), jnp.float32))

    h = _tc0(xpad, W1)
    degb = deg_call(dstp)
    hf = h.reshape(FLAT)
    y1, dinvb = _tc1(hf, degb)
    p = agg_call(y1.reshape(NPAD, CH), srcp, dstp)
    y2 = _tc2(p.reshape(NC, *FLAT), y1, dinvb)
    q = agg_call(y2.reshape(NPAD, CH), srcp, dstp)
    outf = _tc3(q.reshape(NC, *FLAT), y2, dinvb, w2bd, gsum)
    return outf.reshape(NPAD, CH)[:N_NODES]


# speedup vs baseline: 91.3809x; 1.0344x over previous
import functools

import jax
import jax.numpy as jnp
from jax import lax
from jax.experimental import pallas as pl
from jax.experimental.pallas import tpu as pltpu
from jax.experimental.pallas import tpu_sc as plsc

N_NODES = 10000
D_FEAT = 128
CH = 16

NC = 2
NS = 16
NW = NC * NS
B = 128
NPAD = 10240
RPT = NPAD // NS

_MESH = plsc.VectorSubcoreMesh(core_axis_name="c", subcore_axis_name="s")


def _deg_body(dst_hbm, out_hbm, dstbuf, ones_v, zbuf, degbuf, bbuf, deg_sh,
              dsem):
    c = lax.axis_index("c")
    s = lax.axis_index("s")
    w = c * NS + s
    T = dstbuf.shape[0]
    pltpu.sync_copy(dst_hbm.at[pl.ds(w * T, T)], dstbuf)

    z16 = jnp.zeros((16,), jnp.float32)
    o16 = jnp.ones((16,), jnp.float32)

    @pl.loop(0, RPT // 16)
    def _zero(i):
        zbuf[pl.ds(i * 16, 16)] = z16

    @pl.loop(0, B // 16)
    def _ones(i):
        ones_v[pl.ds(i * 16, 16)] = o16

    pltpu.sync_copy(zbuf, deg_sh.at[pl.ds(s * RPT, RPT)])
    plsc.subcore_barrier()

    K = 16

    @pl.loop(0, T, step=K)
    def _accum(jj):
        for b in range(K):
            pltpu.async_copy(ones_v, deg_sh.at[dstbuf.at[jj + b]], dsem,
                             add=True)
        for b in range(K):
            pltpu.make_async_copy(ones_v, deg_sh.at[dstbuf.at[jj + b]],
                                  dsem).wait()

    plsc.subcore_barrier()
    pltpu.sync_copy(deg_sh.at[pl.ds(s * RPT, RPT)], degbuf)

    @pl.loop(0, RPT)
    def _bcast(v):
        val16 = plsc.load_gather(degbuf, [jnp.full((16,), v, jnp.int32)])
        bbuf[v // 8, pl.ds((v % 8) * 16, 16)] = val16

    FR = RPT * CH // 128
    pltpu.sync_copy(bbuf, out_hbm.at[c, pl.ds(s * FR, FR)])


NB = 8


def _agg_body(y_hbm, src_hbm, dst_hbm, out_hbm,
              srcbuf, dstbuf, rows, acc, gsem, ssem):
    c = lax.axis_index("c")
    s = lax.axis_index("s")
    w = c * NS + s
    T = srcbuf.shape[0]
    pltpu.sync_copy(src_hbm.at[pl.ds(w * T, T)], srcbuf)
    pltpu.sync_copy(dst_hbm.at[pl.ds(w * T, T)], dstbuf)

    z16 = jnp.zeros((16,), jnp.float32)

    @pl.loop(0, B)
    def _zero(i):
        rows[0, i] = z16

    @pl.loop(0, RPT // B)
    def _init(i):
        pltpu.sync_copy(rows.at[0], acc.at[pl.ds(s * RPT + i * B, B)])

    plsc.subcore_barrier()

    for b in range(NB):
        pltpu.async_copy(y_hbm.at[srcbuf.at[b]], rows.at[b], gsem.at[b])

    @pl.loop(0, T, step=NB)
    def _edges(jj):
        for b in range(NB):
            j = jj + b
            pltpu.make_async_copy(y_hbm.at[srcbuf.at[j]], rows.at[b],
                                  gsem.at[b]).wait()
            pltpu.async_copy(rows.at[b], acc.at[dstbuf.at[j]], ssem.at[b],
                             add=True)
        for b in range(NB):
            j = jj + b
            pltpu.make_async_copy(rows.at[b], acc.at[dstbuf.at[j]],
                                  ssem.at[b]).wait()

            @pl.when(j + NB < T)
            def _prefetch():
                pltpu.async_copy(y_hbm.at[srcbuf.at[j + NB]], rows.at[b],
                                 gsem.at[b])

    plsc.subcore_barrier()
    pltpu.sync_copy(acc.at[pl.ds(s * RPT, RPT)],
                    out_hbm.at[c, pl.ds(s * RPT, RPT)])


def _make_deg_call(T):
    return pl.kernel(
        _deg_body,
        out_type=jax.ShapeDtypeStruct((NC, NPAD * CH // 128, 128),
                                      jnp.float32),
        mesh=_MESH,
        scratch_types=[
            pltpu.VMEM((T, B), jnp.int32),
            pltpu.VMEM((B,), jnp.float32),
            pltpu.VMEM((RPT,), jnp.float32),
            pltpu.VMEM((RPT,), jnp.float32),
            pltpu.VMEM((RPT * CH // 128, 128), jnp.float32),
            pltpu.VMEM_SHARED((NPAD,), jnp.float32),
            pltpu.SemaphoreType.DMA,
        ],
        compiler_params=pltpu.CompilerParams(needs_layout_passes=False),
    )


def _make_agg_call(T):
    return pl.kernel(
        _agg_body,
        out_type=jax.ShapeDtypeStruct((NC, NPAD, CH), jnp.float32),
        mesh=_MESH,
        scratch_types=[
            pltpu.VMEM((T, B), jnp.int32),
            pltpu.VMEM((T, B), jnp.int32),
            pltpu.VMEM((NB, B, CH), jnp.float32),
            pltpu.VMEM_SHARED((NPAD, CH), jnp.float32),
            pltpu.SemaphoreType.DMA((NB,)),
            pltpu.SemaphoreType.DMA((NB,)),
        ],
        compiler_params=pltpu.CompilerParams(use_tc_tiling_on_sc=False),
    )


_GRID = 8
_RB = NPAD // _GRID
FLAT = (NPAD * CH // 128, 128)


def _tc0_body(x_ref, w1_ref, h_ref):
    h_ref[...] = jnp.dot(x_ref[...], w1_ref[...],
                         preferred_element_type=jnp.float32)


NFR = N_NODES * CH // 128


def _tc1_body(h_ref, degb_ref, y1_ref, dinvb_ref):
    dinvb = 1.0 / jnp.sqrt(degb_ref[0] + degb_ref[1] + 1.0)
    y1_ref[...] = jnp.pad(h_ref[...], ((0, FLAT[0] - NFR), (0, 0))) * dinvb
    dinvb_ref[...] = dinvb


def _tc2_body(p_ref, y1_ref, dinvb_ref, y2_ref):
    agg = (p_ref[0] + p_ref[1] + y1_ref[...]) * dinvb_ref[...]
    y2_ref[...] = jnp.maximum(agg, 0.0) * dinvb_ref[...]


def _tc3_body(q_ref, y2_ref, dinvb_ref, w2bd_ref, gsum_ref, out_ref):
    zf = (q_ref[0] + q_ref[1] + y2_ref[...]) * dinvb_ref[...]
    z = jnp.dot(zf, w2bd_ref[...], preferred_element_type=jnp.float32)
    e = jnp.exp(z - jnp.max(z, axis=-1, keepdims=True))
    denom = jnp.dot(e, gsum_ref[...], preferred_element_type=jnp.float32)
    out_ref[...] = (e / denom)[:NFR]


_tc0 = pl.pallas_call(
    _tc0_body,
    grid=(10,),
    in_specs=[pl.BlockSpec((N_NODES // 10, D_FEAT), lambda i: (i, 0)),
              pl.BlockSpec((D_FEAT, CH), lambda i: (0, 0))],
    out_specs=pl.BlockSpec((N_NODES // 10, CH), lambda i: (i, 0)),
    out_shape=jax.ShapeDtypeStruct((N_NODES, CH), jnp.float32),
)

_tc1 = pl.pallas_call(
    _tc1_body,
    out_shape=[
        jax.ShapeDtypeStruct(FLAT, jnp.float32),
        jax.ShapeDtypeStruct(FLAT, jnp.float32),
    ],
)

_tc2 = pl.pallas_call(
    _tc2_body,
    out_shape=jax.ShapeDtypeStruct(FLAT, jnp.float32),
)

_tc3 = pl.pallas_call(
    _tc3_body,
    out_shape=jax.ShapeDtypeStruct((N_NODES * CH // 128, 128), jnp.float32),
)


def kernel(x, edge_index, W1, W2):
    src = edge_index[0].astype(jnp.int32)
    dst = edge_index[1].astype(jnp.int32)
    E = src.shape[0]
    T = -(-E // (NW * B))
    T = -(-T // NB) * NB
    EPAD = NW * B * T

    pad_idx = N_NODES + (jnp.arange(EPAD - E, dtype=jnp.int32) % (NPAD - N_NODES))
    srcp = jnp.concatenate([src, pad_idx]).reshape(NW * T, B)
    dstp = jnp.concatenate([dst, pad_idx]).reshape(NW * T, B)

    deg_call = _make_deg_call(T)
    agg_call = _make_agg_call(T)

    w2bd = jnp.kron(jnp.eye(8, dtype=jnp.float32), W2)
    gsum = jnp.kron(jnp.eye(8, dtype=jnp.float32),
                    jnp.ones((CH, CH), jnp.float32))

    h = _tc0(x.astype(jnp.float32), W1)
    degb = deg_call(dstp)
    hf = h.reshape(NFR, 128)
    y1, dinvb = _tc1(hf, degb)
    p = agg_call(y1.reshape(NPAD, CH), srcp, dstp)
    y2 = _tc2(p.reshape(NC, *FLAT), y1, dinvb)
    q = agg_call(y2.reshape(NPAD, CH), srcp, dstp)
    outf = _tc3(q.reshape(NC, *FLAT), y2, dinvb, w2bd, gsum)
    return outf.reshape(N_NODES, CH)


# speedup vs baseline: 94.0069x; 1.0287x over previous
import functools

import jax
import jax.numpy as jnp
from jax import lax
from jax.experimental import pallas as pl
from jax.experimental.pallas import tpu as pltpu
from jax.experimental.pallas import tpu_sc as plsc

N_NODES = 10000
D_FEAT = 128
CH = 16

NC = 2
NS = 16
NW = NC * NS
B = 128
NPAD = 10240
RPT = NPAD // NS

_MESH = plsc.VectorSubcoreMesh(core_axis_name="c", subcore_axis_name="s")


def _deg_body(dst_hbm, out_hbm, dstbuf, ones_v, zbuf, degbuf, bbuf, deg_sh,
              dsem, isem):
    c = lax.axis_index("c")
    s = lax.axis_index("s")
    w = c * NS + s
    T = dstbuf.shape[0]
    NCHUNK = dst_hbm.shape[0]
    cnt = NCHUNK // NW + jnp.where(w < NCHUNK % NW, 1, 0)

    @pl.loop(0, T)
    def _ifetch(j):
        @pl.when(j < cnt)
        def _():
            pltpu.async_copy(dst_hbm.at[j * NW + w], dstbuf.at[j], isem)

    z16 = jnp.zeros((16,), jnp.float32)
    o16 = jnp.ones((16,), jnp.float32)

    @pl.loop(0, RPT // 16)
    def _zero(i):
        zbuf[pl.ds(i * 16, 16)] = z16

    @pl.loop(0, B // 16)
    def _ones(i):
        ones_v[pl.ds(i * 16, 16)] = o16

    pltpu.sync_copy(zbuf, deg_sh.at[pl.ds(s * RPT, RPT)])

    @pl.loop(0, T)
    def _idrain(j):
        @pl.when(j < cnt)
        def _():
            pltpu.make_async_copy(dst_hbm.at[j * NW + w], dstbuf.at[j],
                                  isem).wait()

    plsc.subcore_barrier()

    K = 16

    @pl.loop(0, T, step=K)
    def _accum(jj):
        for b in range(K):
            @pl.when(jj + b < cnt)
            def _():
                pltpu.async_copy(ones_v, deg_sh.at[dstbuf.at[jj + b]], dsem,
                                 add=True)
        for b in range(K):
            @pl.when(jj + b < cnt)
            def _():
                pltpu.make_async_copy(ones_v, deg_sh.at[dstbuf.at[jj + b]],
                                      dsem).wait()

    plsc.subcore_barrier()
    pltpu.sync_copy(deg_sh.at[pl.ds(s * RPT, RPT)], degbuf)

    @pl.loop(0, RPT)
    def _bcast(v):
        val16 = plsc.load_gather(degbuf, [jnp.full((16,), v, jnp.int32)])
        bbuf[v // 8, pl.ds((v % 8) * 16, 16)] = val16

    FR = RPT * CH // 128
    pltpu.sync_copy(bbuf, out_hbm.at[c, pl.ds(s * FR, FR)])


NB = 8


def _agg_body(y_hbm, src_hbm, dst_hbm, out_hbm,
              srcbuf, dstbuf, rows, acc, gsem, ssem, isem):
    c = lax.axis_index("c")
    s = lax.axis_index("s")
    w = c * NS + s
    T = srcbuf.shape[0]
    NCHUNK = src_hbm.shape[0]
    cnt = NCHUNK // NW + jnp.where(w < NCHUNK % NW, 1, 0)

    @pl.loop(0, T)
    def _ifetch(j):
        @pl.when(j < cnt)
        def _():
            pltpu.async_copy(src_hbm.at[j * NW + w], srcbuf.at[j], isem)
            pltpu.async_copy(dst_hbm.at[j * NW + w], dstbuf.at[j], isem)

    z16 = jnp.zeros((16,), jnp.float32)

    @pl.loop(0, B)
    def _zero(i):
        rows[0, i] = z16

    @pl.loop(0, RPT // B)
    def _init(i):
        pltpu.sync_copy(rows.at[0], acc.at[pl.ds(s * RPT + i * B, B)])

    @pl.loop(0, T)
    def _idrain(j):
        @pl.when(j < cnt)
        def _():
            pltpu.make_async_copy(src_hbm.at[j * NW + w], srcbuf.at[j],
                                  isem).wait()
            pltpu.make_async_copy(dst_hbm.at[j * NW + w], dstbuf.at[j],
                                  isem).wait()

    plsc.subcore_barrier()

    for b in range(NB):
        pltpu.async_copy(y_hbm.at[srcbuf.at[b]], rows.at[b], gsem.at[b])

    @pl.loop(0, T, step=NB)
    def _edges(jj):
        for b in range(NB):
            j = jj + b

            @pl.when(j < cnt)
            def _():
                pltpu.make_async_copy(y_hbm.at[srcbuf.at[j]], rows.at[b],
                                      gsem.at[b]).wait()
                pltpu.async_copy(rows.at[b], acc.at[dstbuf.at[j]], ssem.at[b],
                                 add=True)
        for b in range(NB):
            j = jj + b

            @pl.when(j < cnt)
            def _():
                pltpu.make_async_copy(rows.at[b], acc.at[dstbuf.at[j]],
                                      ssem.at[b]).wait()

            @pl.when(j + NB < cnt)
            def _prefetch():
                pltpu.async_copy(y_hbm.at[srcbuf.at[j + NB]], rows.at[b],
                                 gsem.at[b])

    plsc.subcore_barrier()
    pltpu.sync_copy(acc.at[pl.ds(s * RPT, RPT)],
                    out_hbm.at[c, pl.ds(s * RPT, RPT)])


def _make_deg_call(T):
    return pl.kernel(
        _deg_body,
        out_type=jax.ShapeDtypeStruct((NC, NPAD * CH // 128, 128),
                                      jnp.float32),
        mesh=_MESH,
        scratch_types=[
            pltpu.VMEM((T, B), jnp.int32),
            pltpu.VMEM((B,), jnp.float32),
            pltpu.VMEM((RPT,), jnp.float32),
            pltpu.VMEM((RPT,), jnp.float32),
            pltpu.VMEM((RPT * CH // 128, 128), jnp.float32),
            pltpu.VMEM_SHARED((NPAD,), jnp.float32),
            pltpu.SemaphoreType.DMA,
            pltpu.SemaphoreType.DMA,
        ],
        compiler_params=pltpu.CompilerParams(needs_layout_passes=False),
    )


def _make_agg_call(T):
    return pl.kernel(
        _agg_body,
        out_type=jax.ShapeDtypeStruct((NC, NPAD, CH), jnp.float32),
        mesh=_MESH,
        scratch_types=[
            pltpu.VMEM((T, B), jnp.int32),
            pltpu.VMEM((T, B), jnp.int32),
            pltpu.VMEM((NB, B, CH), jnp.float32),
            pltpu.VMEM_SHARED((NPAD, CH), jnp.float32),
            pltpu.SemaphoreType.DMA((NB,)),
            pltpu.SemaphoreType.DMA((NB,)),
            pltpu.SemaphoreType.DMA,
        ],
        compiler_params=pltpu.CompilerParams(use_tc_tiling_on_sc=False),
    )


_GRID = 8
_RB = NPAD // _GRID
FLAT = (NPAD * CH // 128, 128)


def _tc0_body(x_ref, w1_ref, h_ref):
    h_ref[...] = jnp.dot(x_ref[...], w1_ref[...],
                         preferred_element_type=jnp.float32)


NFR = N_NODES * CH // 128


def _tc1_body(h_ref, degb_ref, y1_ref, dinvb_ref):
    dinvb = 1.0 / jnp.sqrt(degb_ref[0] + degb_ref[1] + 1.0)
    y1_ref[...] = jnp.pad(h_ref[...], ((0, FLAT[0] - NFR), (0, 0))) * dinvb
    dinvb_ref[...] = dinvb


def _tc2_body(p_ref, y1_ref, dinvb_ref, y2_ref):
    agg = (p_ref[0] + p_ref[1] + y1_ref[...]) * dinvb_ref[...]
    y2_ref[...] = jnp.maximum(agg, 0.0) * dinvb_ref[...]


def _tc3_body(q_ref, y2_ref, dinvb_ref, w2bd_ref, gsum_ref, out_ref):
    zf = (q_ref[0] + q_ref[1] + y2_ref[...]) * dinvb_ref[...]
    z = jnp.dot(zf, w2bd_ref[...], preferred_element_type=jnp.float32)
    e = jnp.exp(z - jnp.max(z, axis=-1, keepdims=True))
    denom = jnp.dot(e, gsum_ref[...], preferred_element_type=jnp.float32)
    out_ref[...] = (e / denom)[:NFR]


_tc0 = pl.pallas_call(
    _tc0_body,
    grid=(10,),
    in_specs=[pl.BlockSpec((N_NODES // 10, D_FEAT), lambda i: (i, 0)),
              pl.BlockSpec((D_FEAT, CH), lambda i: (0, 0))],
    out_specs=pl.BlockSpec((N_NODES // 10, CH), lambda i: (i, 0)),
    out_shape=jax.ShapeDtypeStruct((N_NODES, CH), jnp.float32),
)

_tc1 = pl.pallas_call(
    _tc1_body,
    out_shape=[
        jax.ShapeDtypeStruct(FLAT, jnp.float32),
        jax.ShapeDtypeStruct(FLAT, jnp.float32),
    ],
)

_tc2 = pl.pallas_call(
    _tc2_body,
    out_shape=jax.ShapeDtypeStruct(FLAT, jnp.float32),
)

_tc3 = pl.pallas_call(
    _tc3_body,
    out_shape=jax.ShapeDtypeStruct((N_NODES * CH // 128, 128), jnp.float32),
)


def kernel(x, edge_index, W1, W2):
    src = edge_index[0].astype(jnp.int32)
    dst = edge_index[1].astype(jnp.int32)
    E = src.shape[0]
    NCHUNK = E // B
    T = -(-NCHUNK // NW)
    T = -(-T // NB) * NB

    src2d = src.reshape(NCHUNK, B)
    dst2d = dst.reshape(NCHUNK, B)

    deg_call = _make_deg_call(T)
    agg_call = _make_agg_call(T)

    w2bd = jnp.kron(jnp.eye(8, dtype=jnp.float32), W2)
    gsum = jnp.kron(jnp.eye(8, dtype=jnp.float32),
                    jnp.ones((CH, CH), jnp.float32))

    h = _tc0(x.astype(jnp.float32), W1)
    degb = deg_call(dst2d)
    hf = h.reshape(NFR, 128)
    y1, dinvb = _tc1(hf, degb)
    p = agg_call(y1.reshape(NPAD, CH), src2d, dst2d)
    y2 = _tc2(p.reshape(NC, *FLAT), y1, dinvb)
    q = agg_call(y2.reshape(NPAD, CH), src2d, dst2d)
    outf = _tc3(q.reshape(NC, *FLAT), y2, dinvb, w2bd, gsum)
    return outf.reshape(N_NODES, CH)


# speedup vs baseline: 109.9735x; 1.1698x over previous
import functools

import jax
import jax.numpy as jnp
from jax import lax
from jax.experimental import pallas as pl
from jax.experimental.pallas import tpu as pltpu
from jax.experimental.pallas import tpu_sc as plsc

N_NODES = 10000
D_FEAT = 128
CH = 16

NC = 2
NS = 16
NW = NC * NS
B = 128
NPAD = 10240
RPT = NPAD // NS

_MESH = plsc.VectorSubcoreMesh(core_axis_name="c", subcore_axis_name="s")


def _deg_body(ei_hbm, out_hbm, dstbuf, ones_v, zbuf, degbuf, bbuf, deg_sh,
              dsem, isem):
    c = lax.axis_index("c")
    s = lax.axis_index("s")
    w = c * NS + s
    T = dstbuf.shape[0]
    NCHUNK = ei_hbm.shape[0]
    cnt = NCHUNK // NW + jnp.where(w < NCHUNK % NW, 1, 0)

    @pl.loop(0, T)
    def _ifetch(j):
        @pl.when(j < cnt)
        def _():
            pltpu.async_copy(ei_hbm.at[j * NW + w, 1], dstbuf.at[j], isem)

    z16 = jnp.zeros((16,), jnp.float32)
    o16 = jnp.ones((16,), jnp.float32)

    @pl.loop(0, RPT // 16)
    def _zero(i):
        zbuf[pl.ds(i * 16, 16)] = z16

    @pl.loop(0, B // 16)
    def _ones(i):
        ones_v[pl.ds(i * 16, 16)] = o16

    pltpu.sync_copy(zbuf, deg_sh.at[pl.ds(s * RPT, RPT)])

    @pl.loop(0, T)
    def _idrain(j):
        @pl.when(j < cnt)
        def _():
            pltpu.make_async_copy(ei_hbm.at[j * NW + w, 1], dstbuf.at[j],
                                  isem).wait()

    plsc.subcore_barrier()

    K = 16

    @pl.loop(0, T, step=K)
    def _accum(jj):
        for b in range(K):
            @pl.when(jj + b < cnt)
            def _():
                pltpu.async_copy(ones_v, deg_sh.at[dstbuf.at[jj + b]], dsem,
                                 add=True)
        for b in range(K):
            @pl.when(jj + b < cnt)
            def _():
                pltpu.make_async_copy(ones_v, deg_sh.at[dstbuf.at[jj + b]],
                                      dsem).wait()

    plsc.subcore_barrier()
    pltpu.sync_copy(deg_sh.at[pl.ds(s * RPT, RPT)], degbuf)

    @pl.loop(0, RPT)
    def _bcast(v):
        val16 = plsc.load_gather(degbuf, [jnp.full((16,), v, jnp.int32)])
        bbuf[v // 8, pl.ds((v % 8) * 16, 16)] = val16

    FR = RPT * CH // 128
    pltpu.sync_copy(bbuf, out_hbm.at[c, pl.ds(s * FR, FR)])


NB = 8


def _agg_body(y_hbm, ei_hbm, out_hbm,
              srcbuf, dstbuf, rows, acc, gsem, ssem, isem):
    c = lax.axis_index("c")
    s = lax.axis_index("s")
    w = c * NS + s
    T = srcbuf.shape[0]
    NCHUNK = ei_hbm.shape[0]
    cnt = NCHUNK // NW + jnp.where(w < NCHUNK % NW, 1, 0)

    @pl.loop(0, T)
    def _ifetch(j):
        @pl.when(j < cnt)
        def _():
            pltpu.async_copy(ei_hbm.at[j * NW + w, 0], srcbuf.at[j], isem)
            pltpu.async_copy(ei_hbm.at[j * NW + w, 1], dstbuf.at[j], isem)

    z16 = jnp.zeros((16,), jnp.float32)

    @pl.loop(0, B)
    def _zero(i):
        rows[0, i] = z16

    @pl.loop(0, RPT // B)
    def _init(i):
        pltpu.sync_copy(rows.at[0], acc.at[pl.ds(s * RPT + i * B, B)])

    @pl.loop(0, T)
    def _idrain(j):
        @pl.when(j < cnt)
        def _():
            pltpu.make_async_copy(ei_hbm.at[j * NW + w, 0], srcbuf.at[j],
                                  isem).wait()
            pltpu.make_async_copy(ei_hbm.at[j * NW + w, 1], dstbuf.at[j],
                                  isem).wait()

    plsc.subcore_barrier()

    for b in range(NB):
        pltpu.async_copy(y_hbm.at[srcbuf.at[b]], rows.at[b], gsem.at[b])

    @pl.loop(0, T, step=NB)
    def _edges(jj):
        for b in range(NB):
            j = jj + b

            @pl.when(j < cnt)
            def _():
                pltpu.make_async_copy(y_hbm.at[srcbuf.at[j]], rows.at[b],
                                      gsem.at[b]).wait()
                pltpu.async_copy(rows.at[b], acc.at[dstbuf.at[j]], ssem.at[b],
                                 add=True)
        for b in range(NB):
            j = jj + b

            @pl.when(j < cnt)
            def _():
                pltpu.make_async_copy(rows.at[b], acc.at[dstbuf.at[j]],
                                      ssem.at[b]).wait()

            @pl.when(j + NB < cnt)
            def _prefetch():
                pltpu.async_copy(y_hbm.at[srcbuf.at[j + NB]], rows.at[b],
                                 gsem.at[b])

    plsc.subcore_barrier()
    pltpu.sync_copy(acc.at[pl.ds(s * RPT, RPT)],
                    out_hbm.at[c, pl.ds(s * RPT, RPT)])


def _make_deg_call(T):
    return pl.kernel(
        _deg_body,
        out_type=jax.ShapeDtypeStruct((NC, NPAD * CH // 128, 128),
                                      jnp.float32),
        mesh=_MESH,
        scratch_types=[
            pltpu.VMEM((T, B), jnp.int32),
            pltpu.VMEM((B,), jnp.float32),
            pltpu.VMEM((RPT,), jnp.float32),
            pltpu.VMEM((RPT,), jnp.float32),
            pltpu.VMEM((RPT * CH // 128, 128), jnp.float32),
            pltpu.VMEM_SHARED((NPAD,), jnp.float32),
            pltpu.SemaphoreType.DMA,
            pltpu.SemaphoreType.DMA,
        ],
        compiler_params=pltpu.CompilerParams(needs_layout_passes=False),
    )


def _make_agg_call(T):
    return pl.kernel(
        _agg_body,
        out_type=jax.ShapeDtypeStruct((NC, NPAD, CH), jnp.float32),
        mesh=_MESH,
        scratch_types=[
            pltpu.VMEM((T, B), jnp.int32),
            pltpu.VMEM((T, B), jnp.int32),
            pltpu.VMEM((NB, B, CH), jnp.float32),
            pltpu.VMEM_SHARED((NPAD, CH), jnp.float32),
            pltpu.SemaphoreType.DMA((NB,)),
            pltpu.SemaphoreType.DMA((NB,)),
            pltpu.SemaphoreType.DMA,
        ],
        compiler_params=pltpu.CompilerParams(use_tc_tiling_on_sc=False),
    )


_GRID = 8
_RB = NPAD // _GRID
FLAT = (NPAD * CH // 128, 128)


def _tc0_body(x_ref, w1_ref, h_ref):
    h_ref[...] = jnp.dot(x_ref[...], w1_ref[...],
                         preferred_element_type=jnp.float32)


NFR = N_NODES * CH // 128


def _tc1_body(h_ref, degb_ref, y1_ref, dinvb_ref):
    dinvb = 1.0 / jnp.sqrt(degb_ref[0] + degb_ref[1] + 1.0)
    y1_ref[...] = jnp.pad(h_ref[...], ((0, FLAT[0] - NFR), (0, 0))) * dinvb
    dinvb_ref[...] = dinvb


def _tc2_body(p_ref, y1_ref, dinvb_ref, y2_ref):
    agg = (p_ref[0] + p_ref[1] + y1_ref[...]) * dinvb_ref[...]
    y2_ref[...] = jnp.maximum(agg, 0.0) * dinvb_ref[...]


def _tc3_body(q_ref, y2_ref, dinvb_ref, w2bd_ref, gsum_ref, out_ref):
    zf = (q_ref[0] + q_ref[1] + y2_ref[...]) * dinvb_ref[...]
    z = jnp.dot(zf, w2bd_ref[...], preferred_element_type=jnp.float32)
    e = jnp.exp(z - jnp.max(z, axis=-1, keepdims=True))
    denom = jnp.dot(e, gsum_ref[...], preferred_element_type=jnp.float32)
    out_ref[...] = (e / denom)[:NFR]


_tc0 = pl.pallas_call(
    _tc0_body,
    grid=(10,),
    in_specs=[pl.BlockSpec((N_NODES // 10, D_FEAT), lambda i: (i, 0)),
              pl.BlockSpec((D_FEAT, CH), lambda i: (0, 0))],
    out_specs=pl.BlockSpec((N_NODES // 10, CH), lambda i: (i, 0)),
    out_shape=jax.ShapeDtypeStruct((N_NODES, CH), jnp.float32),
)

_tc1 = pl.pallas_call(
    _tc1_body,
    out_shape=[
        jax.ShapeDtypeStruct(FLAT, jnp.float32),
        jax.ShapeDtypeStruct(FLAT, jnp.float32),
    ],
)

_tc2 = pl.pallas_call(
    _tc2_body,
    out_shape=jax.ShapeDtypeStruct(FLAT, jnp.float32),
)

_tc3 = pl.pallas_call(
    _tc3_body,
    out_shape=jax.ShapeDtypeStruct((N_NODES * CH // 128, 128), jnp.float32),
)


def kernel(x, edge_index, W1, W2):
    E = edge_index.shape[1]
    NCHUNK = E // B
    T = -(-NCHUNK // NW)
    T = -(-T // NB) * NB

    ei = jnp.transpose(edge_index.astype(jnp.int32).reshape(2, NCHUNK, B),
                       (1, 0, 2))

    deg_call = _make_deg_call(T)
    agg_call = _make_agg_call(T)

    w2bd = jnp.kron(jnp.eye(8, dtype=jnp.float32), W2)
    gsum = jnp.kron(jnp.eye(8, dtype=jnp.float32),
                    jnp.ones((CH, CH), jnp.float32))

    h = _tc0(x.astype(jnp.float32), W1)
    degb = deg_call(ei)
    hf = h.reshape(NFR, 128)
    y1, dinvb = _tc1(hf, degb)
    p = agg_call(y1.reshape(NPAD, CH), ei)
    y2 = _tc2(p.reshape(NC, *FLAT), y1, dinvb)
    q = agg_call(y2.reshape(NPAD, CH), ei)
    outf = _tc3(q.reshape(NC, *FLAT), y2, dinvb, w2bd, gsum)
    return outf.reshape(N_NODES, CH)
